# Initial kernel scaffold; baseline (speedup 1.0000x reference)
import jax
import jax.numpy as jnp
from jax.experimental import pallas as pl


def kernel(recon_points, gt_points):
    raise NotImplementedError("write your pallas kernel here")


# speedup vs baseline: 22.4174x; 22.4174x over previous
import functools

import jax
import jax.numpy as jnp
from jax.experimental import pallas as pl
from jax.experimental.pallas import tpu as pltpu

_R2 = 0.1 * 0.1
_NS = 16.0


def _end_loss_kernel(p1_ref, p2t_ref, p2last_ref, out_ref):
    b = pl.program_id(0)
    s = pl.program_id(1)

    p1 = p1_ref[0]
    p2t = p2t_ref[0]
    last = p2last_ref[0]

    dot = jax.lax.dot_general(
        p1, p2t, (((1,), (0,)), ((), ())),
        preferred_element_type=jnp.float32)
    d = -2.0 * dot
    d = d + jnp.sum(p1 * p1, axis=1, keepdims=True)
    d = d + jnp.sum(p2t * p2t, axis=0, keepdims=True)

    m = jnp.where(d <= _R2, 1.0, 0.0)

    n = m.shape[1]
    lane = jax.lax.broadcasted_iota(jnp.int32, m.shape, 1)
    c = m
    k = 1
    while k < n:
        c = c + jnp.where(lane >= k, pltpu.roll(c, k, 1), 0.0)
        k *= 2

    w = m * jnp.where(c <= _NS, 1.0, 0.0)
    f = m * jnp.where(c == 1.0, 1.0, 0.0)

    cols = []
    fcols = []
    for cc in range(3):
        row = p2t[cc:cc + 1, :]
        cols.append(jnp.sum(w * row, axis=1, keepdims=True))
        fcols.append(jnp.sum(f * row, axis=1, keepdims=True))
    sel = jnp.concatenate(cols, axis=1)
    fst = jnp.concatenate(fcols, axis=1)

    cnt = jnp.sum(w, axis=1, keepdims=True)
    has = jnp.sum(f, axis=1, keepdims=True)

    sum_vec = sel + (_NS - cnt) * fst + (1.0 - has) * _NS * last - _NS * p1
    dist = jnp.sqrt(jnp.sum(sum_vec * sum_vec, axis=1, keepdims=True))

    partial = jnp.sum(dist, axis=(0, 1), keepdims=True)

    @pl.when((b == 0) & (s == 0))
    def _():
        out_ref[...] = jnp.zeros_like(out_ref)

    out_ref[...] += partial


@jax.jit
def kernel(recon_points, gt_points):
    B, S, C = recon_points.shape
    N = gt_points.shape[1]
    SB = 256

    gt_t = gt_points.transpose(0, 2, 1)
    gt_last = gt_points[:, N - 1:N, :]

    total = pl.pallas_call(
        _end_loss_kernel,
        grid=(B, S // SB),
        in_specs=[
            pl.BlockSpec((1, SB, C), lambda b, s: (b, s, 0)),
            pl.BlockSpec((1, C, N), lambda b, s: (b, 0, 0)),
            pl.BlockSpec((1, 1, C), lambda b, s: (b, 0, 0)),
        ],
        out_specs=pl.BlockSpec((1, 1), lambda b, s: (0, 0)),
        out_shape=jax.ShapeDtypeStruct((1, 1), jnp.float32),
    )(recon_points, gt_t, gt_last)

    mean_dist = total[0, 0] / (B * S)
    return mean_dist / S * 24


# speedup vs baseline: 37.6164x; 1.6780x over previous
import functools

import jax
import jax.numpy as jnp
from jax.experimental import pallas as pl
from jax.experimental.pallas import tpu as pltpu

_R2 = 0.1 * 0.1
_NS = 16.0
_CH = 32


def _end_loss_kernel(p1_ref, p2t_ref, p2last_ref, p4_ref, e_ref, et_ref,
                     out_ref):
    b = pl.program_id(0)
    s = pl.program_id(1)

    p1 = p1_ref[0]
    p2t = p2t_ref[0]
    last = p2last_ref[0]
    p4 = p4_ref[0]
    e = e_ref[...]
    et = et_ref[...]

    n = p2t.shape[1]
    nc = e.shape[1]

    dot = jax.lax.dot_general(
        p1, p2t, (((1,), (0,)), ((), ())),
        preferred_element_type=jnp.float32)
    d = -2.0 * dot
    d = d + jnp.sum(p1 * p1, axis=1, keepdims=True)
    d = d + jnp.sum(p2t * p2t, axis=0, keepdims=True)

    m = jnp.where(d <= _R2, 1.0, 0.0)

    lane = jax.lax.broadcasted_iota(jnp.int32, (1, n), 1)
    sub = lane & (_CH - 1)
    c = m
    k = 1
    while k < _CH:
        mask = jnp.where(sub >= k, 1.0, 0.0)
        c = c + pltpu.roll(c, k, 1) * mask
        k *= 2

    tot = jax.lax.dot_general(
        m, e, (((1,), (0,)), ((), ())),
        preferred_element_type=jnp.float32)
    lane_c = jax.lax.broadcasted_iota(jnp.int32, (1, nc), 1)
    p = tot
    k = 1
    while k < nc:
        maskc = jnp.where(lane_c >= k, 1.0, 0.0)
        p = p + pltpu.roll(p, k, 1) * maskc
        k *= 2
    p_excl = p - tot
    c = c + jax.lax.dot_general(
        p_excl, et, (((1,), (0,)), ((), ())),
        preferred_element_type=jnp.float32)

    w = jnp.where(c <= _NS, m, 0.0)
    f = jnp.where(c == 1.0, m, 0.0)

    sums_w = jax.lax.dot_general(
        w, p4, (((1,), (0,)), ((), ())),
        preferred_element_type=jnp.float32)
    sums_f = jax.lax.dot_general(
        f, p4, (((1,), (0,)), ((), ())),
        preferred_element_type=jnp.float32)

    sel = sums_w[:, 0:3]
    cnt = sums_w[:, 3:4]
    fst = sums_f[:, 0:3]
    has = sums_f[:, 3:4]

    sum_vec = sel + (_NS - cnt) * fst + (1.0 - has) * _NS * last - _NS * p1
    dist = jnp.sqrt(jnp.sum(sum_vec * sum_vec, axis=1, keepdims=True))

    partial = jnp.sum(dist, axis=(0, 1), keepdims=True)

    @pl.when((b == 0) & (s == 0))
    def _():
        out_ref[...] = jnp.zeros_like(out_ref)

    out_ref[...] += partial


@jax.jit
def kernel(recon_points, gt_points):
    B, S, C = recon_points.shape
    N = gt_points.shape[1]
    SB = 256
    NC = N // _CH

    gt_t = gt_points.transpose(0, 2, 1)
    gt_last = gt_points[:, N - 1:N, :]
    p4 = jnp.concatenate(
        [gt_points, jnp.ones((B, N, 1), jnp.float32)], axis=2)
    chunk_id = jnp.arange(N, dtype=jnp.int32) // _CH
    e = (chunk_id[:, None] == jnp.arange(NC, dtype=jnp.int32)[None, :]
         ).astype(jnp.float32)
    et = e.T

    total = pl.pallas_call(
        _end_loss_kernel,
        grid=(B, S // SB),
        in_specs=[
            pl.BlockSpec((1, SB, C), lambda b, s: (b, s, 0)),
            pl.BlockSpec((1, C, N), lambda b, s: (b, 0, 0)),
            pl.BlockSpec((1, 1, C), lambda b, s: (b, 0, 0)),
            pl.BlockSpec((1, N, 4), lambda b, s: (b, 0, 0)),
            pl.BlockSpec((N, NC), lambda b, s: (0, 0)),
            pl.BlockSpec((NC, N), lambda b, s: (0, 0)),
        ],
        out_specs=pl.BlockSpec((1, 1), lambda b, s: (0, 0)),
        out_shape=jax.ShapeDtypeStruct((1, 1), jnp.float32),
    )(recon_points, gt_t, gt_last, p4, e, et)

    mean_dist = total[0, 0] / (B * S)
    return mean_dist / S * 24


# speedup vs baseline: 47.7818x; 1.2702x over previous
import functools

import jax
import jax.numpy as jnp
from jax.experimental import pallas as pl
from jax.experimental.pallas import tpu as pltpu

_R2 = 0.1 * 0.1
_NS = 16.0
_CH = 32


def _end_loss_kernel(p1_ref, p2t_ref, p2last_ref, p4_ref, e_ref, et_ref,
                     out_ref):
    b = pl.program_id(0)
    s = pl.program_id(1)

    p1 = p1_ref[0]
    p2t = p2t_ref[0]
    last = p2last_ref[0]
    p4 = p4_ref[0]
    e = e_ref[...]
    et = et_ref[...]

    n = p2t.shape[1]
    nc = e.shape[1]

    dot = jax.lax.dot_general(
        p1, p2t, (((1,), (0,)), ((), ())),
        preferred_element_type=jnp.float32)
    d = -2.0 * dot
    d = d + jnp.sum(p1 * p1, axis=1, keepdims=True)
    d = d + jnp.sum(p2t * p2t, axis=0, keepdims=True)

    one = jnp.bfloat16(1.0)
    zero = jnp.bfloat16(0.0)
    m = jnp.where(d <= _R2, 1.0, 0.0).astype(jnp.bfloat16)

    lane = jax.lax.broadcasted_iota(jnp.int32, (1, n), 1)
    sub = lane & (_CH - 1)
    c = m
    k = 1
    while k < _CH:
        mask = jnp.where(sub >= k, 1.0, 0.0).astype(jnp.bfloat16)
        c = c + pltpu.roll(c, k, 1) * mask
        k *= 2

    tot = jax.lax.dot_general(
        m, e, (((1,), (0,)), ((), ())),
        preferred_element_type=jnp.float32)
    lane_c = jax.lax.broadcasted_iota(jnp.int32, (1, nc), 1)
    p = tot
    k = 1
    while k < nc:
        maskc = jnp.where(lane_c >= k, 1.0, 0.0)
        p = p + pltpu.roll(p, k, 1) * maskc
        k *= 2
    p_excl = p - tot
    cfull = c + jax.lax.dot_general(
        p_excl, et, (((1,), (0,)), ((), ())),
        preferred_element_type=jnp.float32).astype(jnp.bfloat16)

    w = jnp.where(cfull <= jnp.bfloat16(_NS), m, zero)
    f = jnp.where(cfull == one, m, zero)

    wf = jnp.concatenate([w, f], axis=0)
    sums = jax.lax.dot_general(
        wf, p4, (((1,), (0,)), ((), ())),
        preferred_element_type=jnp.float32)

    sb = p1.shape[0]
    sel = sums[:sb, 0:3]
    cnt = sums[:sb, 3:4]
    fst = sums[sb:, 0:3]
    has = sums[sb:, 3:4]

    sum_vec = sel + (_NS - cnt) * fst + (1.0 - has) * _NS * last - _NS * p1
    dist = jnp.sqrt(jnp.sum(sum_vec * sum_vec, axis=1, keepdims=True))

    partial = jnp.sum(dist, axis=(0, 1), keepdims=True)

    @pl.when((b == 0) & (s == 0))
    def _():
        out_ref[...] = jnp.zeros_like(out_ref)

    out_ref[...] += partial


@jax.jit
def kernel(recon_points, gt_points):
    B, S, C = recon_points.shape
    N = gt_points.shape[1]
    SB = 256
    NC = N // _CH

    gt_t = gt_points.transpose(0, 2, 1)
    gt_last = gt_points[:, N - 1:N, :]
    p4 = jnp.concatenate(
        [gt_points, jnp.ones((B, N, 1), jnp.float32)], axis=2)
    chunk_id = jnp.arange(N, dtype=jnp.int32) // _CH
    ef = (chunk_id[:, None] == jnp.arange(NC, dtype=jnp.int32)[None, :]
          ).astype(jnp.float32)
    e = ef.astype(jnp.bfloat16)
    et = ef.T

    total = pl.pallas_call(
        _end_loss_kernel,
        grid=(B, S // SB),
        in_specs=[
            pl.BlockSpec((1, SB, C), lambda b, s: (b, s, 0)),
            pl.BlockSpec((1, C, N), lambda b, s: (b, 0, 0)),
            pl.BlockSpec((1, 1, C), lambda b, s: (b, 0, 0)),
            pl.BlockSpec((1, N, 4), lambda b, s: (b, 0, 0)),
            pl.BlockSpec((N, NC), lambda b, s: (0, 0)),
            pl.BlockSpec((NC, N), lambda b, s: (0, 0)),
        ],
        out_specs=pl.BlockSpec((1, 1), lambda b, s: (0, 0)),
        out_shape=jax.ShapeDtypeStruct((1, 1), jnp.float32),
    )(recon_points, gt_t, gt_last, p4, e, et)

    mean_dist = total[0, 0] / (B * S)
    return mean_dist / S * 24


# speedup vs baseline: 52.0469x; 1.0893x over previous
import functools

import jax
import jax.numpy as jnp
from jax.experimental import pallas as pl
from jax.experimental.pallas import tpu as pltpu

_R2 = 0.1 * 0.1
_NS = 16.0
_CH = 32


def _end_loss_kernel(p1a_ref, p2a_ref, p2last_ref, p4_ref, e_ref, et_ref,
                     out_ref):
    b = pl.program_id(0)
    s = pl.program_id(1)

    p1a = p1a_ref[0]
    p2a = p2a_ref[0]
    last = p2last_ref[0]
    p4 = p4_ref[0]
    e = e_ref[...]
    et = et_ref[...]

    n = p2a.shape[1]
    nc = e.shape[1]

    d = jax.lax.dot_general(
        p1a, p2a, (((1,), (0,)), ((), ())),
        preferred_element_type=jnp.float32)

    m = jnp.where(d <= _R2, 1.0, 0.0).astype(jnp.bfloat16)

    lane = jax.lax.broadcasted_iota(jnp.int32, (1, n), 1)
    sub = lane & (_CH - 1)
    c = m
    k = 1
    while k < _CH:
        mask = jnp.where(sub >= k, 1.0, 0.0).astype(jnp.bfloat16)
        c = c + pltpu.roll(c, k, 1) * mask
        k *= 2

    tot = jax.lax.dot_general(
        m, e, (((1,), (0,)), ((), ())),
        preferred_element_type=jnp.float32)
    lane_c = jax.lax.broadcasted_iota(jnp.int32, (1, nc), 1)
    p = tot
    k = 1
    while k < nc:
        maskc = jnp.where(lane_c >= k, 1.0, 0.0)
        p = p + pltpu.roll(p, k, 1) * maskc
        k *= 2
    p_excl = p - tot
    cfull = c + jax.lax.dot_general(
        p_excl, et, (((1,), (0,)), ((), ())),
        preferred_element_type=jnp.float32).astype(jnp.bfloat16)

    zero = jnp.bfloat16(0.0)
    w = jnp.where(cfull <= jnp.bfloat16(_NS), m, zero)
    f = jnp.where(cfull == jnp.bfloat16(1.0), m, zero)

    wf = jnp.concatenate([w, f], axis=0)
    sums = jax.lax.dot_general(
        wf, p4, (((1,), (0,)), ((), ())),
        preferred_element_type=jnp.float32)

    sb = p1a.shape[0]
    p1 = p1a[:, 0:3]
    sel = sums[:sb, 0:3]
    cnt = sums[:sb, 3:4]
    fst = sums[sb:, 0:3]
    has = sums[sb:, 3:4]

    sum_vec = sel + (_NS - cnt) * fst + (1.0 - has) * _NS * last - _NS * p1
    dist = jnp.sqrt(jnp.sum(sum_vec * sum_vec, axis=1, keepdims=True))

    partial = jnp.sum(dist, axis=(0, 1), keepdims=True)

    @pl.when((b == 0) & (s == 0))
    def _():
        out_ref[...] = jnp.zeros_like(out_ref)

    out_ref[...] += partial


@jax.jit
def kernel(recon_points, gt_points):
    B, S, C = recon_points.shape
    N = gt_points.shape[1]
    SB = 512
    NC = N // _CH

    ones_s = jnp.ones((B, S, 1), jnp.float32)
    p1a = jnp.concatenate(
        [recon_points,
         jnp.sum(recon_points * recon_points, axis=2, keepdims=True),
         ones_s], axis=2)
    gt_t = gt_points.transpose(0, 2, 1)
    p2a = jnp.concatenate(
        [-2.0 * gt_t,
         jnp.ones((B, 1, N), jnp.float32),
         jnp.sum(gt_t * gt_t, axis=1, keepdims=True)], axis=1)
    gt_last = gt_points[:, N - 1:N, :]
    p4 = jnp.concatenate(
        [gt_points, jnp.ones((B, N, 1), jnp.float32)], axis=2)
    chunk_id = jnp.arange(N, dtype=jnp.int32) // _CH
    ef = (chunk_id[:, None] == jnp.arange(NC, dtype=jnp.int32)[None, :]
          ).astype(jnp.float32)
    e = ef.astype(jnp.bfloat16)
    et = ef.T

    total = pl.pallas_call(
        _end_loss_kernel,
        grid=(B, S // SB),
        in_specs=[
            pl.BlockSpec((1, SB, 5), lambda b, s: (b, s, 0)),
            pl.BlockSpec((1, 5, N), lambda b, s: (b, 0, 0)),
            pl.BlockSpec((1, 1, C), lambda b, s: (b, 0, 0)),
            pl.BlockSpec((1, N, 4), lambda b, s: (b, 0, 0)),
            pl.BlockSpec((N, NC), lambda b, s: (0, 0)),
            pl.BlockSpec((NC, N), lambda b, s: (0, 0)),
        ],
        out_specs=pl.BlockSpec((1, 1), lambda b, s: (0, 0)),
        out_shape=jax.ShapeDtypeStruct((1, 1), jnp.float32),
    )(p1a, p2a, gt_last, p4, e, et)

    mean_dist = total[0, 0] / (B * S)
    return mean_dist / S * 24
